# baseline (speedup 1.0000x reference)
import functools
import math

import jax
import jax.numpy as jnp
from jax import lax
from jax.experimental import pallas as pl
from jax.experimental.pallas import tpu as pltpu
from jax.experimental.pallas import tpu_sc as plsc

D_MODEL = 1024
SCALE = math.sqrt(D_MODEL)

_info = plsc.get_sparse_core_info()
_NC, _NS, _L = _info.num_cores, _info.num_subcores, _info.num_lanes
_NW = _NC * _NS

_CHUNK = 32
_VECS_PER_ROW = D_MODEL // _L


def _emb_body(table_hbm, x_hbm, out_hbm, idx_v, buf_v, sem):
    wid = lax.axis_index("s") * _NC + lax.axis_index("c")
    bpw = x_hbm.shape[0] // _NW
    base = wid * bpw
    pltpu.sync_copy(x_hbm.at[pl.ds(base, bpw)], idx_v)

    def chunk_body(ci, carry):
        pltpu.async_copy(
            table_hbm.at[idx_v.at[pl.ds(ci * _CHUNK, _CHUNK)]], buf_v, sem
        ).wait()

        def scale_body(i, c2):
            r = i // _VECS_PER_ROW
            col = (i % _VECS_PER_ROW) * _L
            buf_v[r, pl.ds(col, _L)] = buf_v[r, pl.ds(col, _L)] * SCALE
            return c2

        lax.fori_loop(0, _CHUNK * _VECS_PER_ROW, scale_body, 0)
        pltpu.sync_copy(buf_v, out_hbm.at[pl.ds(base + ci * _CHUNK, _CHUNK)])
        return carry

    lax.fori_loop(0, bpw // _CHUNK, chunk_body, 0)


def kernel(table, x):
    b = x.size
    xf = x.reshape(b).astype(jnp.int32)
    mesh = plsc.VectorSubcoreMesh(core_axis_name="c", subcore_axis_name="s")
    run = pl.kernel(
        _emb_body,
        out_type=jax.ShapeDtypeStruct((b, D_MODEL), jnp.float32),
        mesh=mesh,
        scratch_types=[
            pltpu.VMEM((b // _NW,), jnp.int32),
            pltpu.VMEM((_CHUNK, D_MODEL), jnp.float32),
            pltpu.SemaphoreType.DMA,
        ],
    )
    out = run(table, xf)
    return out.reshape(x.shape + (D_MODEL,))


# speedup vs baseline: 2.9750x; 2.9750x over previous
import math

import jax
import jax.numpy as jnp
from jax import lax
from jax.experimental import pallas as pl
from jax.experimental.pallas import tpu as pltpu
from jax.experimental.pallas import tpu_sc as plsc

D_MODEL = 1024
SCALE = math.sqrt(D_MODEL)

_info = plsc.get_sparse_core_info()
_NC, _NS, _L = _info.num_cores, _info.num_subcores, _info.num_lanes
_NW = _NC * _NS

_CHUNK = 32
_VECS_PER_ROW = D_MODEL // _L


def _emb_body(table_hbm, x_hbm, out_hbm,
              idx_v, buf0, buf1, gsem0, gsem1, wsem0, wsem1):
    wid = lax.axis_index("s") * _NC + lax.axis_index("c")
    bpw = x_hbm.shape[0] // _NW
    base = wid * bpw
    pltpu.sync_copy(x_hbm.at[pl.ds(base, bpw)], idx_v)
    nchunks = bpw // _CHUNK

    def gather_start(k, buf, sem):
        pltpu.async_copy(table_hbm.at[idx_v.at[pl.ds(k * _CHUNK, _CHUNK)]],
                         buf, sem)

    def gather_wait(k, buf, sem):
        pltpu.make_async_copy(table_hbm.at[idx_v.at[pl.ds(k * _CHUNK, _CHUNK)]],
                              buf, sem).wait()

    def scatter_start(k, buf, sem):
        pltpu.async_copy(buf, out_hbm.at[pl.ds(base + k * _CHUNK, _CHUNK)], sem)

    def scatter_wait(k, buf, sem):
        pltpu.make_async_copy(buf, out_hbm.at[pl.ds(base + k * _CHUNK, _CHUNK)],
                              sem).wait()

    def scale(buf):
        @plsc.parallel_loop(0, _CHUNK, unroll=2)
        def _(r):
            for j in range(_VECS_PER_ROW):
                col = j * _L
                buf[r, pl.ds(col, _L)] = buf[r, pl.ds(col, _L)] * SCALE

    gather_start(0, buf0, gsem0)

    def body(ci, carry):
        k0 = 2 * ci
        k1 = k0 + 1
        gather_wait(k0, buf0, gsem0)

        @pl.when(ci >= 1)
        def _():
            scatter_wait(k0 - 1, buf1, wsem1)

        gather_start(k1, buf1, gsem1)
        scale(buf0)
        scatter_start(k0, buf0, wsem0)

        gather_wait(k1, buf1, gsem1)
        scatter_wait(k0, buf0, wsem0)

        @pl.when(ci < nchunks // 2 - 1)
        def _():
            gather_start(k0 + 2, buf0, gsem0)

        scale(buf1)
        scatter_start(k1, buf1, wsem1)
        return carry

    lax.fori_loop(0, nchunks // 2, body, 0)
    scatter_wait(nchunks - 1, buf1, wsem1)


def kernel(table, x):
    b = x.size
    xf = x.reshape(b).astype(jnp.int32)
    mesh = plsc.VectorSubcoreMesh(core_axis_name="c", subcore_axis_name="s")
    run = pl.kernel(
        _emb_body,
        out_type=jax.ShapeDtypeStruct((b, D_MODEL), jnp.float32),
        mesh=mesh,
        scratch_types=[
            pltpu.VMEM((b // _NW,), jnp.int32),
            pltpu.VMEM((_CHUNK, D_MODEL), jnp.float32),
            pltpu.VMEM((_CHUNK, D_MODEL), jnp.float32),
            pltpu.SemaphoreType.DMA,
            pltpu.SemaphoreType.DMA,
            pltpu.SemaphoreType.DMA,
            pltpu.SemaphoreType.DMA,
        ],
    )
    out = run(table, xf)
    return out.reshape(x.shape + (D_MODEL,))
